# Initial kernel scaffold; baseline (speedup 1.0000x reference)
import functools

import jax
import jax.numpy as jnp
from jax import lax
from jax.experimental import pallas as pl
from jax.experimental.pallas import tpu as pltpu
from jax.experimental.pallas import tpu_sc as plsc

N = 10000
E = 320000
D = 128
HID = 64
WACC = 144
NC = 2
NS = 16
NW = NC * NS
EPT = E // NW
G = 16
NGRP = EPT // G
RPT = N // NS


def _sc_edge_body(h_hbm, asrc_hbm, adst_hbm, src_hbm, dst_hbm, zeros_hbm,
                  out_hbm,
                  acc, as_v, ad_v, src_v, dst_v, rows_v, stage_v,
                  sidx_v, didx_v, ex_v, gsem):
    c = lax.axis_index("c")
    s = lax.axis_index("s")
    wid = c * NS + s

    pltpu.sync_copy(zeros_hbm, acc.at[pl.ds(s * RPT, RPT)])
    pltpu.sync_copy(asrc_hbm, as_v)
    pltpu.sync_copy(adst_hbm, ad_v)
    ebase = wid * EPT
    pltpu.sync_copy(src_hbm.at[pl.ds(ebase, EPT)], src_v)
    pltpu.sync_copy(dst_hbm.at[pl.ds(ebase, EPT)], dst_v)
    pltpu.sync_copy(zeros_hbm.at[pl.ds(0, G)], stage_v)
    plsc.subcore_barrier()

    def grp(g, carry):
        base = g * G
        for j in range(G // 16):
            off = base + j * 16
            sv = src_v[pl.ds(off, 16)]
            dv = dst_v[pl.ds(off, 16)]
            sidx_v[pl.ds(j * 16, 16)] = sv
            didx_v[pl.ds(j * 16, 16)] = dv
            a = plsc.load_gather(as_v, [sv]) + plsc.load_gather(ad_v, [dv])
            t = jnp.where(a >= 0.0, a, 0.2 * a)
            ex = jnp.exp(t)
            ex_v[pl.ds(j * 16, 16)] = ex
            lane = lax.iota(jnp.int32, 16) + j * 16
            plsc.store_scatter(stage_v,
                               [lane, jnp.full((16,), D, jnp.int32)], ex)
        pltpu.async_copy(h_hbm.at[sidx_v], rows_v, gsem).wait()
        for e in range(G):
            w = ex_v[e]
            for k in range(D // 16):
                stage_v[e, pl.ds(k * 16, 16)] = rows_v[e, pl.ds(k * 16, 16)] * w
        pltpu.sync_copy(stage_v, acc.at[didx_v], add=True)
        return carry

    lax.fori_loop(0, NGRP, grp, 0)
    plsc.subcore_barrier()
    pltpu.sync_copy(acc.at[pl.ds(s * RPT, RPT)],
                    out_hbm.at[c, pl.ds(s * RPT, RPT)])


_sc_edge = functools.partial(
    pl.kernel,
    out_type=jax.ShapeDtypeStruct((NC, N, WACC), jnp.float32),
    mesh=plsc.VectorSubcoreMesh(core_axis_name="c", subcore_axis_name="s"),
    scratch_types=[
        pltpu.VMEM_SHARED((N, WACC), jnp.float32),
        pltpu.VMEM((N,), jnp.float32),
        pltpu.VMEM((N,), jnp.float32),
        pltpu.VMEM((EPT,), jnp.int32),
        pltpu.VMEM((EPT,), jnp.int32),
        pltpu.VMEM((G, D), jnp.float32),
        pltpu.VMEM((G, WACC), jnp.float32),
        pltpu.VMEM((G,), jnp.int32),
        pltpu.VMEM((G,), jnp.int32),
        pltpu.VMEM((G,), jnp.float32),
        pltpu.SemaphoreType.DMA,
    ],
)(_sc_edge_body)


_BLK = 1000
_GRID = N // _BLK


def _tc_embed_body(x_ref, W_ref, asrc_ref, adst_ref, h_ref, s_ref, d_ref):
    h = jnp.dot(x_ref[...], W_ref[...], preferred_element_type=jnp.float32)
    h_ref[...] = h
    s_ref[...] = jnp.sum(h * asrc_ref[...], axis=1, keepdims=True)
    d_ref[...] = jnp.sum(h * adst_ref[...], axis=1, keepdims=True)


def _tc_embed(x, W, asrc, adst):
    return pl.pallas_call(
        _tc_embed_body,
        grid=(_GRID,),
        in_specs=[
            pl.BlockSpec((_BLK, D), lambda i: (i, 0)),
            pl.BlockSpec((D, D), lambda i: (0, 0)),
            pl.BlockSpec((1, D), lambda i: (0, 0)),
            pl.BlockSpec((1, D), lambda i: (0, 0)),
        ],
        out_specs=[
            pl.BlockSpec((_BLK, D), lambda i: (i, 0)),
            pl.BlockSpec((_BLK, 1), lambda i: (i, 0)),
            pl.BlockSpec((_BLK, 1), lambda i: (i, 0)),
        ],
        out_shape=[
            jax.ShapeDtypeStruct((N, D), jnp.float32),
            jax.ShapeDtypeStruct((N, 1), jnp.float32),
            jax.ShapeDtypeStruct((N, 1), jnp.float32),
        ],
    )(x, W, asrc, adst)


def _tc_combine_body(acc_ref, b_ref, W_ref, asrc_ref, adst_ref,
                     h_ref, s_ref, d_ref):
    a = acc_ref[0] + acc_ref[1]
    num = a[:, :D]
    den = a[:, D:D + 1]
    res = num / (den + 1e-16) + b_ref[...]
    hin = jnp.maximum(res, 0.0)
    h = jnp.dot(hin, W_ref[...], preferred_element_type=jnp.float32)
    h_ref[...] = h
    s_ref[...] = jnp.sum(h * asrc_ref[...], axis=1, keepdims=True)
    d_ref[...] = jnp.sum(h * adst_ref[...], axis=1, keepdims=True)


def _tc_combine(acc, b, W, asrc, adst):
    return pl.pallas_call(
        _tc_combine_body,
        grid=(_GRID,),
        in_specs=[
            pl.BlockSpec((NC, _BLK, WACC), lambda i: (0, i, 0)),
            pl.BlockSpec((1, D), lambda i: (0, 0)),
            pl.BlockSpec((D, D), lambda i: (0, 0)),
            pl.BlockSpec((1, D), lambda i: (0, 0)),
            pl.BlockSpec((1, D), lambda i: (0, 0)),
        ],
        out_specs=[
            pl.BlockSpec((_BLK, D), lambda i: (i, 0)),
            pl.BlockSpec((_BLK, 1), lambda i: (i, 0)),
            pl.BlockSpec((_BLK, 1), lambda i: (i, 0)),
        ],
        out_shape=[
            jax.ShapeDtypeStruct((N, D), jnp.float32),
            jax.ShapeDtypeStruct((N, 1), jnp.float32),
            jax.ShapeDtypeStruct((N, 1), jnp.float32),
        ],
    )(acc, b, W, asrc, adst)


def _tc_head_body(acc_ref, b_ref, lw1_ref, lb1_ref, lw2_ref, lb2_ref, o_ref):
    a = acc_ref[0] + acc_ref[1]
    num = a[:, :D]
    den = a[:, D:D + 1]
    res = num / (den + 1e-16) + b_ref[...]
    t = jnp.dot(res, lw1_ref[...], preferred_element_type=jnp.float32)
    t = jnp.maximum(t + lb1_ref[...], 0.0)
    o = jnp.dot(t, lw2_ref[...], preferred_element_type=jnp.float32)
    o_ref[...] = o + lb2_ref[...]


def _tc_head(acc, b, lw1, lb1, lw2, lb2):
    return pl.pallas_call(
        _tc_head_body,
        grid=(_GRID,),
        in_specs=[
            pl.BlockSpec((NC, _BLK, WACC), lambda i: (0, i, 0)),
            pl.BlockSpec((1, D), lambda i: (0, 0)),
            pl.BlockSpec((D, HID), lambda i: (0, 0)),
            pl.BlockSpec((1, HID), lambda i: (0, 0)),
            pl.BlockSpec((HID, 1), lambda i: (0, 0)),
            pl.BlockSpec((1, 1), lambda i: (0, 0)),
        ],
        out_specs=pl.BlockSpec((_BLK, 1), lambda i: (i, 0)),
        out_shape=jax.ShapeDtypeStruct((N, 1), jnp.float32),
    )(acc, b, lw1, lb1, lw2, lb2)


def kernel(x, edge_index, W1, a_src1, a_dst1, b1, W2, a_src2, a_dst2, b2,
           lw1, lb1, lw2, lb2):
    src = edge_index[0].astype(jnp.int32)
    dst = edge_index[1].astype(jnp.int32)
    zeros = jnp.zeros((RPT, WACC), jnp.float32)

    h1, as1, ad1 = _tc_embed(x, W1, a_src1, a_dst1)
    acc1 = _sc_edge(h1, as1.reshape(N), ad1.reshape(N), src, dst, zeros)
    h2, as2, ad2 = _tc_combine(acc1, b1.reshape(1, D), W2, a_src2, a_dst2)
    acc2 = _sc_edge(h2, as2.reshape(N), ad2.reshape(N), src, dst, zeros)
    return _tc_head(acc2, b2.reshape(1, D), lw1, lb1.reshape(1, HID),
                    lw2, lb2.reshape(1, 1))


# speedup vs baseline: 15.5869x; 15.5869x over previous
import functools

import jax
import jax.numpy as jnp
from jax import lax
from jax.experimental import pallas as pl
from jax.experimental.pallas import tpu as pltpu
from jax.experimental.pallas import tpu_sc as plsc

N = 10000
E = 320000
D = 128
HID = 64
NC = 2
NS = 16
NW = NC * NS
EPT = E // NW
G = 16
NP = 10112
RPT = NP // NS
EC = 2000
NCH = EPT // EC
NGRP = EC // G


def _sc_edge_body(h_hbm, asrc_hbm, adst_hbm, src_hbm, dst_hbm, zeros_hbm,
                  zn_hbm,
                  out_hbm, outd_hbm,
                  acc, as_v, ad_v, src_v, dst_v, rows_v, stage_v,
                  dtab_v, sidx_v, didx_v, gsem):
    c = lax.axis_index("c")
    s = lax.axis_index("s")
    wid = c * NS + s

    pltpu.sync_copy(zeros_hbm, acc.at[pl.ds(s * RPT, RPT)])
    pltpu.sync_copy(asrc_hbm, as_v)
    pltpu.sync_copy(adst_hbm, ad_v)
    pltpu.sync_copy(zn_hbm, dtab_v)
    plsc.subcore_barrier()

    lanes = lax.iota(jnp.int32, 16)
    ebase = wid * EPT

    def grp(g, carry):
        base = g * G
        exs = []
        for j in range(G // 16):
            off = base + j * 16
            sv = src_v[pl.ds(off, 16)]
            dv = dst_v[pl.ds(off, 16)]
            sidx_v[pl.ds(j * 16, 16)] = sv
            didx_v[pl.ds(j * 16, 16)] = dv
            a = plsc.load_gather(as_v, [sv]) + plsc.load_gather(ad_v, [dv])
            t = jnp.where(a >= 0.0, a, 0.2 * a)
            ex = jnp.exp(t)
            exs.append(ex)
            for li in range(16):
                plsc.addupdate_scatter(dtab_v, [dv], ex, mask=lanes == li)
        pltpu.async_copy(h_hbm.at[sidx_v], rows_v, gsem).wait()
        for e in range(G):
            w = exs[e // 16][e % 16]
            for k in range(D // 16):
                stage_v[e, pl.ds(k * 16, 16)] = rows_v[e, pl.ds(k * 16, 16)] * w
        pltpu.sync_copy(stage_v, acc.at[didx_v], add=True)
        return carry

    for ch in range(NCH):
        pltpu.sync_copy(src_hbm.at[pl.ds(ebase + ch * EC, EC)], src_v)
        pltpu.sync_copy(dst_hbm.at[pl.ds(ebase + ch * EC, EC)], dst_v)
        lax.fori_loop(0, NGRP, grp, 0)

    pltpu.sync_copy(dtab_v, outd_hbm.at[wid])
    plsc.subcore_barrier()
    pltpu.sync_copy(acc.at[pl.ds(s * RPT, RPT)],
                    out_hbm.at[c, pl.ds(s * RPT, RPT)])


@functools.cache
def _sc_edge_kernel():
    return functools.partial(
        pl.kernel,
        out_type=(jax.ShapeDtypeStruct((NC, NP, D), jnp.float32),
                  jax.ShapeDtypeStruct((NW, N), jnp.float32)),
        mesh=plsc.VectorSubcoreMesh(core_axis_name="c", subcore_axis_name="s",
                                    num_cores=NC, num_subcores=NS),
        scratch_types=[
            pltpu.VMEM_SHARED((NP, D), jnp.float32),
            pltpu.VMEM((N,), jnp.float32),
            pltpu.VMEM((N,), jnp.float32),
            pltpu.VMEM((EC,), jnp.int32),
            pltpu.VMEM((EC,), jnp.int32),
            pltpu.VMEM((G, D), jnp.float32),
            pltpu.VMEM((G, D), jnp.float32),
            pltpu.VMEM((N,), jnp.float32),
            pltpu.VMEM((G,), jnp.int32),
            pltpu.VMEM((G,), jnp.int32),
            pltpu.SemaphoreType.DMA,
        ],
        compiler_params=pltpu.CompilerParams(needs_layout_passes=False),
    )(_sc_edge_body)


def _sc_edge(*args):
    return _sc_edge_kernel()(*args)


def _tc_embed_body(x_ref, W_ref, asrc_ref, adst_ref, h_ref, s_ref, d_ref):
    h = jnp.dot(x_ref[...], W_ref[...], preferred_element_type=jnp.float32)
    h_ref[...] = h
    s_ref[...] = jnp.sum(h * asrc_ref[...], axis=1, keepdims=True)
    d_ref[...] = jnp.sum(h * adst_ref[...], axis=1, keepdims=True)


def _tc_embed(x, W, asrc, adst):
    return pl.pallas_call(
        _tc_embed_body,
        out_shape=[
            jax.ShapeDtypeStruct((N, D), jnp.float32),
            jax.ShapeDtypeStruct((N, 1), jnp.float32),
            jax.ShapeDtypeStruct((N, 1), jnp.float32),
        ],
    )(x, W, asrc, adst)


def _den_col(den_blk):
    ones = jnp.ones((NW, 1), jnp.float32)
    return jax.lax.dot_general(den_blk, ones, (((0,), (0,)), ((), ())),
                               preferred_element_type=jnp.float32)


def _tc_combine_body(acc_ref, den_ref, b_ref, W_ref, asrc_ref, adst_ref,
                     h_ref, s_ref, d_ref):
    num = acc_ref[0, :N] + acc_ref[1, :N]
    den = _den_col(den_ref[...])
    res = num / (den + 1e-16) + b_ref[...]
    hin = jnp.maximum(res, 0.0)
    h = jnp.dot(hin, W_ref[...], preferred_element_type=jnp.float32)
    h_ref[...] = h
    s_ref[...] = jnp.sum(h * asrc_ref[...], axis=1, keepdims=True)
    d_ref[...] = jnp.sum(h * adst_ref[...], axis=1, keepdims=True)


def _tc_combine(acc, den, b, W, asrc, adst):
    return pl.pallas_call(
        _tc_combine_body,
        out_shape=[
            jax.ShapeDtypeStruct((N, D), jnp.float32),
            jax.ShapeDtypeStruct((N, 1), jnp.float32),
            jax.ShapeDtypeStruct((N, 1), jnp.float32),
        ],
    )(acc, den, b, W, asrc, adst)


def _tc_head_body(acc_ref, den_ref, b_ref, lw1_ref, lb1_ref, lw2_ref,
                  lb2_ref, o_ref):
    num = acc_ref[0, :N] + acc_ref[1, :N]
    den = _den_col(den_ref[...])
    res = num / (den + 1e-16) + b_ref[...]
    t = jnp.dot(res, lw1_ref[...], preferred_element_type=jnp.float32)
    t = jnp.maximum(t + lb1_ref[...], 0.0)
    o = jnp.dot(t, lw2_ref[...], preferred_element_type=jnp.float32)
    o_ref[...] = o + lb2_ref[...]


def _tc_head(acc, den, b, lw1, lb1, lw2, lb2):
    return pl.pallas_call(
        _tc_head_body,
        out_shape=jax.ShapeDtypeStruct((N, 1), jnp.float32),
    )(acc, den, b, lw1, lb1, lw2, lb2)


def kernel(x, edge_index, W1, a_src1, a_dst1, b1, W2, a_src2, a_dst2, b2,
           lw1, lb1, lw2, lb2):
    src = edge_index[0].astype(jnp.int32)
    dst = edge_index[1].astype(jnp.int32)
    zeros = jnp.zeros((RPT, D), jnp.float32)
    zn = jnp.zeros((N,), jnp.float32)

    h1, as1, ad1 = _tc_embed(x, W1, a_src1, a_dst1)
    acc1, den1 = _sc_edge(h1, as1.reshape(N), ad1.reshape(N), src, dst,
                          zeros, zn)
    h2, as2, ad2 = _tc_combine(acc1, den1, b1.reshape(1, D), W2,
                               a_src2, a_dst2)
    acc2, den2 = _sc_edge(h2, as2.reshape(N), ad2.reshape(N), src, dst,
                          zeros, zn)
    return _tc_head(acc2, den2, b2.reshape(1, D),
                    lw1, lb1.reshape(1, HID), lw2, lb2.reshape(1, 1))


# speedup vs baseline: 30.2584x; 1.9413x over previous
import functools

import jax
import jax.numpy as jnp
from jax import lax
from jax.experimental import pallas as pl
from jax.experimental.pallas import tpu as pltpu
from jax.experimental.pallas import tpu_sc as plsc

N = 10000
E = 320000
D = 128
HID = 64
NC = 2
NS = 16
NW = NC * NS
EPT = E // NW
G = 16
NP = 10112
RPT = NP // NS
EC = 2000
NCH = EPT // EC
NGRP = EC // G


def _sc_edge_body(h_hbm, asrc_hbm, adst_hbm, src_hbm, dst_hbm, zeros_hbm,
                  zn_hbm,
                  out_hbm, outd_hbm,
                  acc, as_v, ad_v, src_v, dst_v, rows_v, stage_v,
                  dtab_v, dummy_v, sidx_v, didx_v,
                  gsem0, gsem1, ssem0, ssem1):
    c = lax.axis_index("c")
    s = lax.axis_index("s")
    wid = c * NS + s
    gsems = (gsem0, gsem1)
    ssems = (ssem0, ssem1)

    pltpu.sync_copy(zeros_hbm, acc.at[pl.ds(s * RPT, RPT)])
    pltpu.sync_copy(asrc_hbm, as_v)
    pltpu.sync_copy(adst_hbm, ad_v)
    pltpu.sync_copy(zn_hbm, dtab_v)
    plsc.subcore_barrier()

    lanes = lax.iota(jnp.int32, 16)
    ebase = wid * EPT

    def prep_fire(off, slot):
        sidx_v[slot, pl.ds(0, 16)] = src_v[pl.ds(off, 16)]
        pltpu.async_copy(h_hbm.at[sidx_v.at[slot]], rows_v.at[slot],
                         gsems[slot])

    def process(off, slot):
        pltpu.make_async_copy(h_hbm.at[sidx_v.at[slot]], rows_v.at[slot],
                              gsems[slot]).wait()
        sv = sidx_v[slot, pl.ds(0, 16)]
        dv = dst_v[pl.ds(off, 16)]
        a = plsc.load_gather(as_v, [sv]) + plsc.load_gather(ad_v, [dv])
        t = jnp.where(a >= 0.0, a, 0.2 * a)
        ex = jnp.exp(t)
        for li in range(16):
            plsc.addupdate_scatter(dtab_v, [dv], ex, mask=lanes == li)
        pltpu.make_async_copy(stage_v.at[slot], acc.at[didx_v.at[slot]],
                              ssems[slot]).wait()
        didx_v[slot, pl.ds(0, 16)] = dv
        for e in range(16):
            w = ex[e]
            for k in range(D // 16):
                stage_v[slot, e, pl.ds(k * 16, 16)] = (
                    rows_v[slot, e, pl.ds(k * 16, 16)] * w)
        pltpu.async_copy(stage_v.at[slot], acc.at[didx_v.at[slot]],
                         ssems[slot], add=True)

    for ch in range(NCH):
        pltpu.sync_copy(src_hbm.at[pl.ds(ebase + ch * EC, EC)], src_v)
        pltpu.sync_copy(dst_hbm.at[pl.ds(ebase + ch * EC, EC)], dst_v)
        pltpu.async_copy(zeros_hbm.at[pl.ds(0, 16)], dummy_v, ssem0)
        pltpu.async_copy(zeros_hbm.at[pl.ds(0, 16)], dummy_v, ssem1)
        prep_fire(0, 0)

        def body(g2, carry):
            a0 = g2 * 2
            prep_fire((a0 + 1) * G, 1)
            process(a0 * G, 0)
            prep_fire((a0 + 2) * G, 0)
            process((a0 + 1) * G, 1)
            return carry

        lax.fori_loop(0, (NGRP - 1) // 2, body, 0)
        process((NGRP - 1) * G, 0)
        pltpu.make_async_copy(stage_v.at[0], acc.at[didx_v.at[0]],
                              ssem0).wait()
        pltpu.make_async_copy(stage_v.at[1], acc.at[didx_v.at[1]],
                              ssem1).wait()

    pltpu.sync_copy(dtab_v, outd_hbm.at[wid])
    plsc.subcore_barrier()
    pltpu.sync_copy(acc.at[pl.ds(s * RPT, RPT)],
                    out_hbm.at[c, pl.ds(s * RPT, RPT)])


@functools.cache
def _sc_edge_kernel():
    return functools.partial(
        pl.kernel,
        out_type=(jax.ShapeDtypeStruct((NC, NP, D), jnp.float32),
                  jax.ShapeDtypeStruct((NW, N), jnp.float32)),
        mesh=plsc.VectorSubcoreMesh(core_axis_name="c", subcore_axis_name="s",
                                    num_cores=NC, num_subcores=NS),
        scratch_types=[
            pltpu.VMEM_SHARED((NP, D), jnp.float32),
            pltpu.VMEM((N,), jnp.float32),
            pltpu.VMEM((N,), jnp.float32),
            pltpu.VMEM((EC,), jnp.int32),
            pltpu.VMEM((EC,), jnp.int32),
            pltpu.VMEM((2, G, D), jnp.float32),
            pltpu.VMEM((2, G, D), jnp.float32),
            pltpu.VMEM((N,), jnp.float32),
            pltpu.VMEM((G, D), jnp.float32),
            pltpu.VMEM((2, G), jnp.int32),
            pltpu.VMEM((2, G), jnp.int32),
            pltpu.SemaphoreType.DMA,
            pltpu.SemaphoreType.DMA,
            pltpu.SemaphoreType.DMA,
            pltpu.SemaphoreType.DMA,
        ],
        compiler_params=pltpu.CompilerParams(needs_layout_passes=False),
    )(_sc_edge_body)


def _sc_edge(*args):
    return _sc_edge_kernel()(*args)


def _tc_embed_body(x_ref, W_ref, asrc_ref, adst_ref, h_ref, s_ref, d_ref):
    h = jnp.dot(x_ref[...], W_ref[...], preferred_element_type=jnp.float32)
    h_ref[...] = h
    s_ref[...] = jnp.sum(h * asrc_ref[...], axis=1, keepdims=True)
    d_ref[...] = jnp.sum(h * adst_ref[...], axis=1, keepdims=True)


def _tc_embed(x, W, asrc, adst):
    return pl.pallas_call(
        _tc_embed_body,
        out_shape=[
            jax.ShapeDtypeStruct((N, D), jnp.float32),
            jax.ShapeDtypeStruct((N, 1), jnp.float32),
            jax.ShapeDtypeStruct((N, 1), jnp.float32),
        ],
    )(x, W, asrc, adst)


def _den_col(den_blk):
    ones = jnp.ones((NW, 1), jnp.float32)
    return jax.lax.dot_general(den_blk, ones, (((0,), (0,)), ((), ())),
                               preferred_element_type=jnp.float32)


def _tc_combine_body(acc_ref, den_ref, b_ref, W_ref, asrc_ref, adst_ref,
                     h_ref, s_ref, d_ref):
    num = acc_ref[0, :N] + acc_ref[1, :N]
    den = _den_col(den_ref[...])
    res = num / (den + 1e-16) + b_ref[...]
    hin = jnp.maximum(res, 0.0)
    h = jnp.dot(hin, W_ref[...], preferred_element_type=jnp.float32)
    h_ref[...] = h
    s_ref[...] = jnp.sum(h * asrc_ref[...], axis=1, keepdims=True)
    d_ref[...] = jnp.sum(h * adst_ref[...], axis=1, keepdims=True)


def _tc_combine(acc, den, b, W, asrc, adst):
    return pl.pallas_call(
        _tc_combine_body,
        out_shape=[
            jax.ShapeDtypeStruct((N, D), jnp.float32),
            jax.ShapeDtypeStruct((N, 1), jnp.float32),
            jax.ShapeDtypeStruct((N, 1), jnp.float32),
        ],
    )(acc, den, b, W, asrc, adst)


def _tc_head_body(acc_ref, den_ref, b_ref, lw1_ref, lb1_ref, lw2_ref,
                  lb2_ref, o_ref):
    num = acc_ref[0, :N] + acc_ref[1, :N]
    den = _den_col(den_ref[...])
    res = num / (den + 1e-16) + b_ref[...]
    t = jnp.dot(res, lw1_ref[...], preferred_element_type=jnp.float32)
    t = jnp.maximum(t + lb1_ref[...], 0.0)
    o = jnp.dot(t, lw2_ref[...], preferred_element_type=jnp.float32)
    o_ref[...] = o + lb2_ref[...]


def _tc_head(acc, den, b, lw1, lb1, lw2, lb2):
    return pl.pallas_call(
        _tc_head_body,
        out_shape=jax.ShapeDtypeStruct((N, 1), jnp.float32),
    )(acc, den, b, lw1, lb1, lw2, lb2)


def kernel(x, edge_index, W1, a_src1, a_dst1, b1, W2, a_src2, a_dst2, b2,
           lw1, lb1, lw2, lb2):
    src = edge_index[0].astype(jnp.int32)
    dst = edge_index[1].astype(jnp.int32)
    zeros = jnp.zeros((RPT, D), jnp.float32)
    zn = jnp.zeros((N,), jnp.float32)

    h1, as1, ad1 = _tc_embed(x, W1, a_src1, a_dst1)
    acc1, den1 = _sc_edge(h1, as1.reshape(N), ad1.reshape(N), src, dst,
                          zeros, zn)
    h2, as2, ad2 = _tc_combine(acc1, den1, b1.reshape(1, D), W2,
                               a_src2, a_dst2)
    acc2, den2 = _sc_edge(h2, as2.reshape(N), ad2.reshape(N), src, dst,
                          zeros, zn)
    return _tc_head(acc2, den2, b2.reshape(1, D),
                    lw1, lb1.reshape(1, HID), lw2, lb2.reshape(1, 1))


# speedup vs baseline: 37.8548x; 1.2511x over previous
import functools

import jax
import jax.numpy as jnp
from jax import lax
from jax.experimental import pallas as pl
from jax.experimental.pallas import tpu as pltpu
from jax.experimental.pallas import tpu_sc as plsc

N = 10000
E = 320000
D = 128
HID = 64
NC = 2
NS = 16
NW = NC * NS
EPT = E // NW
G = 16
NP = 10112
RPT = NP // NS
EC = 2000
NCH = EPT // EC
NGRP = EC // G


def _sc_edge_body(h_hbm, asrc_hbm, adst_hbm, src_hbm, dst_hbm, zeros_hbm,
                  zn_hbm,
                  out_hbm, outd_hbm,
                  acc, as_v, ad_v, src_v, dst_v, rows_v,
                  dtab_v, sidx_v, didx_v,
                  gsem0, gsem1, gsem2, gsem3, ssem0, ssem1, ssem2, ssem3):
    c = lax.axis_index("c")
    s = lax.axis_index("s")
    wid = c * NS + s
    gsems = (gsem0, gsem1, gsem2, gsem3)
    ssems = (ssem0, ssem1, ssem2, ssem3)

    pltpu.sync_copy(zeros_hbm, acc.at[pl.ds(s * RPT, RPT)])
    pltpu.sync_copy(asrc_hbm, as_v)
    pltpu.sync_copy(adst_hbm, ad_v)
    pltpu.sync_copy(zn_hbm, dtab_v)
    plsc.subcore_barrier()

    lanes = lax.iota(jnp.int32, 16)
    ebase = wid * EPT

    def prep_fire(off, slot):
        pltpu.make_async_copy(rows_v.at[slot], acc.at[didx_v.at[slot]],
                              ssems[slot]).wait()
        sidx_v[slot, pl.ds(0, 16)] = src_v[pl.ds(off, 16)]
        pltpu.async_copy(h_hbm.at[sidx_v.at[slot]], rows_v.at[slot],
                         gsems[slot])

    def process(off, slot):
        sv = sidx_v[slot, pl.ds(0, 16)]
        dv = dst_v[pl.ds(off, 16)]
        a = plsc.load_gather(as_v, [sv]) + plsc.load_gather(ad_v, [dv])
        t = jnp.where(a >= 0.0, a, 0.2 * a)
        ex = jnp.exp(t)
        for li in range(16):
            plsc.addupdate_scatter(dtab_v, [dv], ex, mask=lanes == li)
        pltpu.make_async_copy(h_hbm.at[sidx_v.at[slot]], rows_v.at[slot],
                              gsems[slot]).wait()
        didx_v[slot, pl.ds(0, 16)] = dv
        for e in range(16):
            w = ex[e]
            for k in range(D // 16):
                rows_v[slot, e, pl.ds(k * 16, 16)] = (
                    rows_v[slot, e, pl.ds(k * 16, 16)] * w)
        pltpu.async_copy(rows_v.at[slot], acc.at[didx_v.at[slot]],
                         ssems[slot], add=True)

    for sl in range(4):
        pltpu.async_copy(zeros_hbm.at[pl.ds(0, 16)], rows_v.at[sl],
                         ssems[sl])

    def chunk(ch, carry):
        pltpu.sync_copy(src_hbm.at[pl.ds(ebase + ch * EC, EC)], src_v)
        pltpu.sync_copy(dst_hbm.at[pl.ds(ebase + ch * EC, EC)], dst_v)
        for g in range(3):
            prep_fire(g * G, g)

        def body(q, qcarry):
            g0 = q * 4
            for u in range(4):
                process((g0 + u) * G, u)
                prep_fire((g0 + u + 3) * G, (u + 3) % 4)
            return qcarry

        lax.fori_loop(0, (NGRP - 5) // 4, body, 0)
        for g in range(NGRP - 5, NGRP):
            process(g * G, g % 4)
            if g + 3 < NGRP:
                prep_fire((g + 3) * G, (g + 3) % 4)
        return carry

    lax.fori_loop(0, NCH, chunk, 0)
    for sl in range(4):
        pltpu.make_async_copy(rows_v.at[sl], acc.at[didx_v.at[sl]],
                              ssems[sl]).wait()

    pltpu.sync_copy(dtab_v, outd_hbm.at[wid])
    plsc.subcore_barrier()
    pltpu.sync_copy(acc.at[pl.ds(s * RPT, RPT)],
                    out_hbm.at[c, pl.ds(s * RPT, RPT)])


@functools.cache
def _sc_edge_kernel():
    return functools.partial(
        pl.kernel,
        out_type=(jax.ShapeDtypeStruct((NC, NP, D), jnp.float32),
                  jax.ShapeDtypeStruct((NW, N), jnp.float32)),
        mesh=plsc.VectorSubcoreMesh(core_axis_name="c", subcore_axis_name="s",
                                    num_cores=NC, num_subcores=NS),
        scratch_types=[
            pltpu.VMEM_SHARED((NP, D), jnp.float32),
            pltpu.VMEM((N,), jnp.float32),
            pltpu.VMEM((N,), jnp.float32),
            pltpu.VMEM((EC,), jnp.int32),
            pltpu.VMEM((EC,), jnp.int32),
            pltpu.VMEM((4, G, D), jnp.float32),
            pltpu.VMEM((N,), jnp.float32),
            pltpu.VMEM((4, G), jnp.int32),
            pltpu.VMEM((4, G), jnp.int32),
            pltpu.SemaphoreType.DMA,
            pltpu.SemaphoreType.DMA,
            pltpu.SemaphoreType.DMA,
            pltpu.SemaphoreType.DMA,
            pltpu.SemaphoreType.DMA,
            pltpu.SemaphoreType.DMA,
            pltpu.SemaphoreType.DMA,
            pltpu.SemaphoreType.DMA,
        ],
        compiler_params=pltpu.CompilerParams(needs_layout_passes=False),
    )(_sc_edge_body)


def _sc_edge(*args):
    return _sc_edge_kernel()(*args)


def _tc_embed_body(x_ref, W_ref, asrc_ref, adst_ref, h_ref, s_ref, d_ref):
    h = jnp.dot(x_ref[...], W_ref[...], preferred_element_type=jnp.float32,
                precision=jax.lax.Precision.HIGHEST)
    h_ref[...] = h
    s_ref[...] = jnp.sum(h * asrc_ref[...], axis=1, keepdims=True)
    d_ref[...] = jnp.sum(h * adst_ref[...], axis=1, keepdims=True)


def _tc_embed(x, W, asrc, adst):
    return pl.pallas_call(
        _tc_embed_body,
        out_shape=[
            jax.ShapeDtypeStruct((N, D), jnp.float32),
            jax.ShapeDtypeStruct((N, 1), jnp.float32),
            jax.ShapeDtypeStruct((N, 1), jnp.float32),
        ],
    )(x, W, asrc, adst)


def _den_col(den_blk):
    ones = jnp.ones((NW, 1), jnp.float32)
    return jax.lax.dot_general(den_blk, ones, (((0,), (0,)), ((), ())),
                               preferred_element_type=jnp.float32,
                               precision=jax.lax.Precision.HIGHEST)


def _tc_combine_body(acc_ref, den_ref, b_ref, W_ref, asrc_ref, adst_ref,
                     h_ref, s_ref, d_ref):
    num = acc_ref[0, :N] + acc_ref[1, :N]
    den = _den_col(den_ref[...])
    res = num / (den + 1e-16) + b_ref[...]
    hin = jnp.maximum(res, 0.0)
    h = jnp.dot(hin, W_ref[...], preferred_element_type=jnp.float32,
                precision=jax.lax.Precision.HIGHEST)
    h_ref[...] = h
    s_ref[...] = jnp.sum(h * asrc_ref[...], axis=1, keepdims=True)
    d_ref[...] = jnp.sum(h * adst_ref[...], axis=1, keepdims=True)


def _tc_combine(acc, den, b, W, asrc, adst):
    return pl.pallas_call(
        _tc_combine_body,
        out_shape=[
            jax.ShapeDtypeStruct((N, D), jnp.float32),
            jax.ShapeDtypeStruct((N, 1), jnp.float32),
            jax.ShapeDtypeStruct((N, 1), jnp.float32),
        ],
    )(acc, den, b, W, asrc, adst)


def _tc_head_body(acc_ref, den_ref, b_ref, lw1_ref, lb1_ref, lw2_ref,
                  lb2_ref, o_ref):
    num = acc_ref[0, :N] + acc_ref[1, :N]
    den = _den_col(den_ref[...])
    res = num / (den + 1e-16) + b_ref[...]
    t = jnp.dot(res, lw1_ref[...], preferred_element_type=jnp.float32,
                precision=jax.lax.Precision.HIGHEST)
    t = jnp.maximum(t + lb1_ref[...], 0.0)
    o = jnp.dot(t, lw2_ref[...], preferred_element_type=jnp.float32,
                precision=jax.lax.Precision.HIGHEST)
    o_ref[...] = o + lb2_ref[...]


def _tc_head(acc, den, b, lw1, lb1, lw2, lb2):
    return pl.pallas_call(
        _tc_head_body,
        out_shape=jax.ShapeDtypeStruct((N, 1), jnp.float32),
    )(acc, den, b, lw1, lb1, lw2, lb2)


def kernel(x, edge_index, W1, a_src1, a_dst1, b1, W2, a_src2, a_dst2, b2,
           lw1, lb1, lw2, lb2):
    src = edge_index[0].astype(jnp.int32)
    dst = edge_index[1].astype(jnp.int32)
    zeros = jnp.zeros((RPT, D), jnp.float32)
    zn = jnp.zeros((N,), jnp.float32)

    h1, as1, ad1 = _tc_embed(x, W1, a_src1, a_dst1)
    acc1, den1 = _sc_edge(h1, as1.reshape(N), ad1.reshape(N), src, dst,
                          zeros, zn)
    h2, as2, ad2 = _tc_combine(acc1, den1, b1.reshape(1, D), W2,
                               a_src2, a_dst2)
    acc2, den2 = _sc_edge(h2, as2.reshape(N), ad2.reshape(N), src, dst,
                          zeros, zn)
    return _tc_head(acc2, den2, b2.reshape(1, D),
                    lw1, lb1.reshape(1, HID), lw2, lb2.reshape(1, 1))


# speedup vs baseline: 40.3380x; 1.0656x over previous
import functools

import jax
import jax.numpy as jnp
from jax import lax
from jax.experimental import pallas as pl
from jax.experimental.pallas import tpu as pltpu
from jax.experimental.pallas import tpu_sc as plsc

N = 10000
E = 320000
D = 128
HID = 64
NC = 2
NS = 16
NW = NC * NS
EPT = E // NW
G = 16
NP = 10112
RPT = NP // NS
EC = 2000
NCH = EPT // EC
NGRP = EC // G


def _sc_edge_body(h_hbm, asrc_hbm, adst_hbm, src_hbm, dst_hbm, zeros_hbm,
                  zn_hbm,
                  out_hbm, outd_hbm,
                  acc, as_v, ad_v, src_v, dst_v, rows_v,
                  dtab_v, sidx_v, didx_v,
                  gsem0, gsem1, gsem2, gsem3, ssem0, ssem1, ssem2, ssem3):
    c = lax.axis_index("c")
    s = lax.axis_index("s")
    wid = c * NS + s
    gsems = (gsem0, gsem1, gsem2, gsem3)
    ssems = (ssem0, ssem1, ssem2, ssem3)

    pltpu.sync_copy(zeros_hbm, acc.at[pl.ds(s * RPT, RPT)])
    pltpu.sync_copy(asrc_hbm, as_v)
    pltpu.sync_copy(adst_hbm, ad_v)
    pltpu.sync_copy(zn_hbm, dtab_v)
    plsc.subcore_barrier()

    lanes = lax.iota(jnp.int32, 16)
    ebase = wid * EPT

    def prep_fire(off, slot):
        pltpu.make_async_copy(rows_v.at[slot], acc.at[didx_v.at[slot]],
                              ssems[slot]).wait()
        sidx_v[slot, pl.ds(0, 16)] = src_v[pl.ds(off, 16)]
        pltpu.async_copy(h_hbm.at[sidx_v.at[slot]], rows_v.at[slot],
                         gsems[slot])

    def process(off, slot):
        sv = sidx_v[slot, pl.ds(0, 16)]
        dv = dst_v[pl.ds(off, 16)]
        a = plsc.load_gather(as_v, [sv]) + plsc.load_gather(ad_v, [dv])
        t = jnp.where(a >= 0.0, a, 0.2 * a)
        ex = jnp.exp(t)
        for li in range(16):
            plsc.addupdate_scatter(dtab_v, [dv], ex, mask=lanes == li)
        pltpu.make_async_copy(h_hbm.at[sidx_v.at[slot]], rows_v.at[slot],
                              gsems[slot]).wait()
        didx_v[slot, pl.ds(0, 16)] = dv
        for e in range(16):
            w = ex[e]
            for k in range(D // 16):
                rows_v[slot, e, pl.ds(k * 16, 16)] = (
                    rows_v[slot, e, pl.ds(k * 16, 16)] * w)
        pltpu.async_copy(rows_v.at[slot], acc.at[didx_v.at[slot]],
                         ssems[slot], add=True)

    for sl in range(4):
        pltpu.async_copy(zeros_hbm.at[pl.ds(0, 16)], rows_v.at[sl],
                         ssems[sl])

    def chunk(ch, carry):
        pltpu.sync_copy(src_hbm.at[pl.ds(ebase + ch * EC, EC)], src_v)
        pltpu.sync_copy(dst_hbm.at[pl.ds(ebase + ch * EC, EC)], dst_v)
        for g in range(3):
            prep_fire(g * G, g)

        def body(q, qcarry):
            g0 = q * 4
            for u in range(4):
                process((g0 + u) * G, u)
                prep_fire((g0 + u + 3) * G, (u + 3) % 4)
            return qcarry

        lax.fori_loop(0, (NGRP - 5) // 4, body, 0)
        for g in range(NGRP - 5, NGRP):
            process(g * G, g % 4)
            if g + 3 < NGRP:
                prep_fire((g + 3) * G, (g + 3) % 4)
        return carry

    lax.fori_loop(0, NCH, chunk, 0)
    for sl in range(4):
        pltpu.make_async_copy(rows_v.at[sl], acc.at[didx_v.at[sl]],
                              ssems[sl]).wait()

    pltpu.sync_copy(dtab_v, outd_hbm.at[wid])
    plsc.subcore_barrier()
    pltpu.sync_copy(acc.at[pl.ds(s * RPT, RPT)],
                    out_hbm.at[c, pl.ds(s * RPT, RPT)])


@functools.cache
def _sc_edge_kernel():
    return functools.partial(
        pl.kernel,
        out_type=(jax.ShapeDtypeStruct((NC, NP, D), jnp.float32),
                  jax.ShapeDtypeStruct((NW, N), jnp.float32)),
        mesh=plsc.VectorSubcoreMesh(core_axis_name="c", subcore_axis_name="s",
                                    num_cores=NC, num_subcores=NS),
        scratch_types=[
            pltpu.VMEM_SHARED((NP, D), jnp.float32),
            pltpu.VMEM((N,), jnp.float32),
            pltpu.VMEM((N,), jnp.float32),
            pltpu.VMEM((EC,), jnp.int32),
            pltpu.VMEM((EC,), jnp.int32),
            pltpu.VMEM((4, G, D), jnp.float32),
            pltpu.VMEM((N,), jnp.float32),
            pltpu.VMEM((4, G), jnp.int32),
            pltpu.VMEM((4, G), jnp.int32),
            pltpu.SemaphoreType.DMA,
            pltpu.SemaphoreType.DMA,
            pltpu.SemaphoreType.DMA,
            pltpu.SemaphoreType.DMA,
            pltpu.SemaphoreType.DMA,
            pltpu.SemaphoreType.DMA,
            pltpu.SemaphoreType.DMA,
            pltpu.SemaphoreType.DMA,
        ],
        compiler_params=pltpu.CompilerParams(needs_layout_passes=False),
    )(_sc_edge_body)


def _sc_edge(*args):
    return _sc_edge_kernel()(*args)


def _tc_embed_body(x_ref, W_ref, asrc_ref, adst_ref, h_ref, s_ref, d_ref):
    h = jnp.dot(x_ref[...], W_ref[...], preferred_element_type=jnp.float32)
    h_ref[...] = h
    s_ref[...] = jnp.sum(h * asrc_ref[...], axis=1, keepdims=True)
    d_ref[...] = jnp.sum(h * adst_ref[...], axis=1, keepdims=True)


def _tc_embed(x, W, asrc, adst):
    return pl.pallas_call(
        _tc_embed_body,
        out_shape=[
            jax.ShapeDtypeStruct((N, D), jnp.float32),
            jax.ShapeDtypeStruct((N, 1), jnp.float32),
            jax.ShapeDtypeStruct((N, 1), jnp.float32),
        ],
    )(x, W, asrc, adst)


def _den_col(den_blk):
    ones = jnp.ones((NW, 1), jnp.float32)
    return jax.lax.dot_general(den_blk, ones, (((0,), (0,)), ((), ())),
                               preferred_element_type=jnp.float32)


def _tc_combine_body(acc_ref, den_ref, b_ref, W_ref, asrc_ref, adst_ref,
                     h_ref, s_ref, d_ref):
    num = acc_ref[0, :N] + acc_ref[1, :N]
    den = _den_col(den_ref[...])
    res = num / (den + 1e-16) + b_ref[...]
    hin = jnp.maximum(res, 0.0)
    h = jnp.dot(hin, W_ref[...], preferred_element_type=jnp.float32)
    h_ref[...] = h
    s_ref[...] = jnp.sum(h * asrc_ref[...], axis=1, keepdims=True)
    d_ref[...] = jnp.sum(h * adst_ref[...], axis=1, keepdims=True)


def _tc_combine(acc, den, b, W, asrc, adst):
    return pl.pallas_call(
        _tc_combine_body,
        out_shape=[
            jax.ShapeDtypeStruct((N, D), jnp.float32),
            jax.ShapeDtypeStruct((N, 1), jnp.float32),
            jax.ShapeDtypeStruct((N, 1), jnp.float32),
        ],
    )(acc, den, b, W, asrc, adst)


def _tc_head_body(acc_ref, den_ref, b_ref, lw1_ref, lb1_ref, lw2_ref,
                  lb2_ref, o_ref):
    num = acc_ref[0, :N] + acc_ref[1, :N]
    den = _den_col(den_ref[...])
    res = num / (den + 1e-16) + b_ref[...]
    t = jnp.dot(res, lw1_ref[...], preferred_element_type=jnp.float32)
    t = jnp.maximum(t + lb1_ref[...], 0.0)
    o = jnp.dot(t, lw2_ref[...], preferred_element_type=jnp.float32)
    o_ref[...] = o + lb2_ref[...]


def _tc_head(acc, den, b, lw1, lb1, lw2, lb2):
    return pl.pallas_call(
        _tc_head_body,
        out_shape=jax.ShapeDtypeStruct((N, 1), jnp.float32),
    )(acc, den, b, lw1, lb1, lw2, lb2)


def kernel(x, edge_index, W1, a_src1, a_dst1, b1, W2, a_src2, a_dst2, b2,
           lw1, lb1, lw2, lb2):
    src = edge_index[0].astype(jnp.int32)
    dst = edge_index[1].astype(jnp.int32)
    zeros = jnp.zeros((RPT, D), jnp.float32)
    zn = jnp.zeros((N,), jnp.float32)

    h1, as1, ad1 = _tc_embed(x, W1, a_src1, a_dst1)
    acc1, den1 = _sc_edge(h1, as1.reshape(N), ad1.reshape(N), src, dst,
                          zeros, zn)
    h2, as2, ad2 = _tc_combine(acc1, den1, b1.reshape(1, D), W2,
                               a_src2, a_dst2)
    acc2, den2 = _sc_edge(h2, as2.reshape(N), ad2.reshape(N), src, dst,
                          zeros, zn)
    return _tc_head(acc2, den2, b2.reshape(1, D),
                    lw1, lb1.reshape(1, HID), lw2, lb2.reshape(1, 1))


# speedup vs baseline: 51.2404x; 1.2703x over previous
import functools

import jax
import jax.numpy as jnp
from jax import lax
from jax.experimental import pallas as pl
from jax.experimental.pallas import tpu as pltpu
from jax.experimental.pallas import tpu_sc as plsc

N = 10000
E = 320000
D = 128
HID = 64
NC = 2
NS = 16
NW = NC * NS
EPT = E // NW
G = 16
NP = 10112
RPT = NP // NS
EC = 2000
NCH = EPT // EC
NGRP = EC // G
NBUF = 6


def _sc_edge_body(h_hbm, asrc_hbm, adst_hbm, src_hbm, dst_hbm, zeros_hbm,
                  zn_hbm,
                  out_hbm, outd_hbm,
                  acc, as_v, ad_v, src_v, dst_v, rows_v,
                  dtab_v, sidx_v, didx_v,
                  gsem0, gsem1, gsem2, gsem3, gsem4, gsem5,
                  ssem0, ssem1, ssem2, ssem3, ssem4, ssem5):
    c = lax.axis_index("c")
    s = lax.axis_index("s")
    wid = c * NS + s
    gsems = (gsem0, gsem1, gsem2, gsem3, gsem4, gsem5)
    ssems = (ssem0, ssem1, ssem2, ssem3, ssem4, ssem5)

    pltpu.sync_copy(zeros_hbm, acc.at[pl.ds(s * RPT, RPT)])
    pltpu.sync_copy(asrc_hbm, as_v)
    pltpu.sync_copy(adst_hbm, ad_v)
    pltpu.sync_copy(zn_hbm, dtab_v)
    plsc.subcore_barrier()

    lanes = lax.iota(jnp.int32, 16)
    ebase = wid * EPT

    def prep_fire(off, slot):
        pltpu.make_async_copy(rows_v.at[slot], acc.at[didx_v.at[slot]],
                              ssems[slot]).wait()
        sidx_v[slot, pl.ds(0, 16)] = src_v[pl.ds(off, 16)]
        pltpu.async_copy(h_hbm.at[sidx_v.at[slot]], rows_v.at[slot],
                         gsems[slot])

    def process(off, slot):
        sv = sidx_v[slot, pl.ds(0, 16)]
        dv = dst_v[pl.ds(off, 16)]
        a = plsc.load_gather(as_v, [sv]) + plsc.load_gather(ad_v, [dv])
        t = jnp.where(a >= 0.0, a, 0.2 * a)
        ex = jnp.exp(t)
        for li in range(16):
            plsc.addupdate_scatter(dtab_v, [dv], ex, mask=lanes == li)
        pltpu.make_async_copy(h_hbm.at[sidx_v.at[slot]], rows_v.at[slot],
                              gsems[slot]).wait()
        didx_v[slot, pl.ds(0, 16)] = dv
        for e in range(16):
            w = ex[e]
            for k in range(D // 16):
                rows_v[slot, e, pl.ds(k * 16, 16)] = (
                    rows_v[slot, e, pl.ds(k * 16, 16)] * w)
        pltpu.async_copy(rows_v.at[slot], acc.at[didx_v.at[slot]],
                         ssems[slot], add=True)

    for sl in range(NBUF):
        pltpu.async_copy(zeros_hbm.at[pl.ds(0, 16)], rows_v.at[sl],
                         ssems[sl])

    def chunk(ch, carry):
        pltpu.sync_copy(src_hbm.at[pl.ds(ebase + ch * EC, EC)], src_v)
        pltpu.sync_copy(dst_hbm.at[pl.ds(ebase + ch * EC, EC)], dst_v)
        for g in range(NBUF - 1):
            prep_fire(g * G, g)

        def body(q, qcarry):
            g0 = q * NBUF
            for u in range(NBUF):
                process((g0 + u) * G, u)
                prep_fire((g0 + u + NBUF - 1) * G, (u + NBUF - 1) % NBUF)
            return qcarry

        nsteady = (NGRP - (NBUF - 1)) // NBUF * NBUF
        lax.fori_loop(0, nsteady // NBUF, body, 0)
        for g in range(nsteady, NGRP):
            process(g * G, g % NBUF)
            if g + NBUF - 1 < NGRP:
                prep_fire((g + NBUF - 1) * G, (g + NBUF - 1) % NBUF)
        return carry

    lax.fori_loop(0, NCH, chunk, 0)
    for sl in range(NBUF):
        pltpu.make_async_copy(rows_v.at[sl], acc.at[didx_v.at[sl]],
                              ssems[sl]).wait()

    pltpu.sync_copy(dtab_v, outd_hbm.at[wid])
    plsc.subcore_barrier()
    pltpu.sync_copy(acc.at[pl.ds(s * RPT, RPT)],
                    out_hbm.at[c, pl.ds(s * RPT, RPT)])


@functools.cache
def _sc_edge_kernel():
    return functools.partial(
        pl.kernel,
        out_type=(jax.ShapeDtypeStruct((NC, NP, D), jnp.float32),
                  jax.ShapeDtypeStruct((NW, N), jnp.float32)),
        mesh=plsc.VectorSubcoreMesh(core_axis_name="c", subcore_axis_name="s",
                                    num_cores=NC, num_subcores=NS),
        scratch_types=[
            pltpu.VMEM_SHARED((NP, D), jnp.float32),
            pltpu.VMEM((N,), jnp.float32),
            pltpu.VMEM((N,), jnp.float32),
            pltpu.VMEM((EC,), jnp.int32),
            pltpu.VMEM((EC,), jnp.int32),
            pltpu.VMEM((NBUF, G, D), jnp.float32),
            pltpu.VMEM((N,), jnp.float32),
            pltpu.VMEM((NBUF, G), jnp.int32),
            pltpu.VMEM((NBUF, G), jnp.int32),
        ] + [pltpu.SemaphoreType.DMA] * (2 * NBUF),
        compiler_params=pltpu.CompilerParams(needs_layout_passes=False),
    )(_sc_edge_body)


def _sc_edge(*args):
    return _sc_edge_kernel()(*args)


def _tc_embed_body(x_ref, W_ref, asrc_ref, adst_ref, h_ref, s_ref, d_ref):
    h = jnp.dot(x_ref[...], W_ref[...], preferred_element_type=jnp.float32)
    h_ref[...] = h
    s_ref[...] = jnp.sum(h * asrc_ref[...], axis=1, keepdims=True)
    d_ref[...] = jnp.sum(h * adst_ref[...], axis=1, keepdims=True)


def _tc_embed(x, W, asrc, adst):
    return pl.pallas_call(
        _tc_embed_body,
        out_shape=[
            jax.ShapeDtypeStruct((N, D), jnp.float32),
            jax.ShapeDtypeStruct((N, 1), jnp.float32),
            jax.ShapeDtypeStruct((N, 1), jnp.float32),
        ],
    )(x, W, asrc, adst)


def _den_col(den_blk):
    ones = jnp.ones((NW, 1), jnp.float32)
    return jax.lax.dot_general(den_blk, ones, (((0,), (0,)), ((), ())),
                               preferred_element_type=jnp.float32)


def _tc_combine_body(acc_ref, den_ref, b_ref, W_ref, asrc_ref, adst_ref,
                     h_ref, s_ref, d_ref):
    num = acc_ref[0, :N] + acc_ref[1, :N]
    den = _den_col(den_ref[...])
    res = num / (den + 1e-16) + b_ref[...]
    hin = jnp.maximum(res, 0.0)
    h = jnp.dot(hin, W_ref[...], preferred_element_type=jnp.float32)
    h_ref[...] = h
    s_ref[...] = jnp.sum(h * asrc_ref[...], axis=1, keepdims=True)
    d_ref[...] = jnp.sum(h * adst_ref[...], axis=1, keepdims=True)


def _tc_combine(acc, den, b, W, asrc, adst):
    return pl.pallas_call(
        _tc_combine_body,
        out_shape=[
            jax.ShapeDtypeStruct((N, D), jnp.float32),
            jax.ShapeDtypeStruct((N, 1), jnp.float32),
            jax.ShapeDtypeStruct((N, 1), jnp.float32),
        ],
    )(acc, den, b, W, asrc, adst)


def _tc_head_body(acc_ref, den_ref, b_ref, lw1_ref, lb1_ref, lw2_ref,
                  lb2_ref, o_ref):
    num = acc_ref[0, :N] + acc_ref[1, :N]
    den = _den_col(den_ref[...])
    res = num / (den + 1e-16) + b_ref[...]
    t = jnp.dot(res, lw1_ref[...], preferred_element_type=jnp.float32)
    t = jnp.maximum(t + lb1_ref[...], 0.0)
    o = jnp.dot(t, lw2_ref[...], preferred_element_type=jnp.float32)
    o_ref[...] = o + lb2_ref[...]


def _tc_head(acc, den, b, lw1, lb1, lw2, lb2):
    return pl.pallas_call(
        _tc_head_body,
        out_shape=jax.ShapeDtypeStruct((N, 1), jnp.float32),
    )(acc, den, b, lw1, lb1, lw2, lb2)


def kernel(x, edge_index, W1, a_src1, a_dst1, b1, W2, a_src2, a_dst2, b2,
           lw1, lb1, lw2, lb2):
    src = edge_index[0].astype(jnp.int32)
    dst = edge_index[1].astype(jnp.int32)
    zeros = jnp.zeros((RPT, D), jnp.float32)
    zn = jnp.zeros((N,), jnp.float32)

    h1, as1, ad1 = _tc_embed(x, W1, a_src1, a_dst1)
    acc1, den1 = _sc_edge(h1, as1.reshape(N), ad1.reshape(N), src, dst,
                          zeros, zn)
    h2, as2, ad2 = _tc_combine(acc1, den1, b1.reshape(1, D), W2,
                               a_src2, a_dst2)
    acc2, den2 = _sc_edge(h2, as2.reshape(N), ad2.reshape(N), src, dst,
                          zeros, zn)
    return _tc_head(acc2, den2, b2.reshape(1, D),
                    lw1, lb1.reshape(1, HID), lw2, lb2.reshape(1, 1))


# speedup vs baseline: 52.5877x; 1.0263x over previous
import functools

import jax
import jax.numpy as jnp
from jax import lax
from jax.experimental import pallas as pl
from jax.experimental.pallas import tpu as pltpu
from jax.experimental.pallas import tpu_sc as plsc

N = 10000
E = 320000
D = 128
HID = 64
NC = 2
NS = 16
NW = NC * NS
EPT = E // NW
G = 16
NP = 10112
RPT = NP // NS
EC = 2000
NCH = EPT // EC
NGRP = EC // G
NBUF = 7


def _sc_edge_body(h_hbm, asrc_hbm, adst_hbm, src_hbm, dst_hbm, zeros_hbm,
                  zn_hbm,
                  out_hbm, outd_hbm,
                  acc, as_v, ad_v, src_v, dst_v, rows_v, dtab_v,
                  gsem0, gsem1, gsem2, gsem3, gsem4, gsem5, gsem6,
                  ssem0, ssem1, ssem2, ssem3, ssem4, ssem5, ssem6):
    c = lax.axis_index("c")
    s = lax.axis_index("s")
    wid = c * NS + s
    gsems = (gsem0, gsem1, gsem2, gsem3, gsem4, gsem5, gsem6)
    ssems = (ssem0, ssem1, ssem2, ssem3, ssem4, ssem5, ssem6)
    zidx = lax.iota(jnp.int32, 16)

    pltpu.sync_copy(zeros_hbm, acc.at[pl.ds(s * RPT, RPT)])
    pltpu.sync_copy(asrc_hbm, as_v)
    pltpu.sync_copy(adst_hbm, ad_v)
    pltpu.sync_copy(zn_hbm, dtab_v)
    plsc.subcore_barrier()

    lanes = lax.iota(jnp.int32, 16)
    ebase = wid * EPT

    def prep_fire(off, slot):
        pltpu.make_async_copy(rows_v.at[slot], acc.at[zidx],
                              ssems[slot]).wait()
        pltpu.async_copy(h_hbm.at[src_v[pl.ds(off, 16)]], rows_v.at[slot],
                         gsems[slot])

    def process(off, slot):
        sv = src_v[pl.ds(off, 16)]
        dv = dst_v[pl.ds(off, 16)]
        a = plsc.load_gather(as_v, [sv]) + plsc.load_gather(ad_v, [dv])
        t = jnp.where(a >= 0.0, a, 0.2 * a)
        ex = jnp.exp(t)
        for li in range(16):
            plsc.addupdate_scatter(dtab_v, [dv], ex, mask=lanes == li)
        pltpu.make_async_copy(h_hbm.at[zidx], rows_v.at[slot],
                              gsems[slot]).wait()
        for e in range(16):
            w = ex[e]
            for k in range(D // 16):
                rows_v[slot, e, pl.ds(k * 16, 16)] = (
                    rows_v[slot, e, pl.ds(k * 16, 16)] * w)
        pltpu.async_copy(rows_v.at[slot], acc.at[dv], ssems[slot],
                         add=True)

    for sl in range(NBUF):
        pltpu.async_copy(zeros_hbm.at[pl.ds(0, 16)], rows_v.at[sl],
                         ssems[sl])

    def chunk(ch, carry):
        pltpu.sync_copy(src_hbm.at[pl.ds(ebase + ch * EC, EC)], src_v)
        pltpu.sync_copy(dst_hbm.at[pl.ds(ebase + ch * EC, EC)], dst_v)
        for g in range(NBUF - 1):
            prep_fire(g * G, g)

        def body(q, qcarry):
            g0 = q * NBUF
            for u in range(NBUF):
                process((g0 + u) * G, u)
                prep_fire((g0 + u + NBUF - 1) * G, (u + NBUF - 1) % NBUF)
            return qcarry

        nsteady = (NGRP - (NBUF - 1)) // NBUF * NBUF
        lax.fori_loop(0, nsteady // NBUF, body, 0)
        for g in range(nsteady, NGRP):
            process(g * G, g % NBUF)
            if g + NBUF - 1 < NGRP:
                prep_fire((g + NBUF - 1) * G, (g + NBUF - 1) % NBUF)
        return carry

    lax.fori_loop(0, NCH, chunk, 0)
    for sl in range(NBUF):
        pltpu.make_async_copy(rows_v.at[sl], acc.at[zidx],
                              ssems[sl]).wait()

    pltpu.sync_copy(dtab_v, outd_hbm.at[wid])
    plsc.subcore_barrier()
    pltpu.sync_copy(acc.at[pl.ds(s * RPT, RPT)],
                    out_hbm.at[c, pl.ds(s * RPT, RPT)])


@functools.cache
def _sc_edge_kernel():
    return functools.partial(
        pl.kernel,
        out_type=(jax.ShapeDtypeStruct((NC, NP, D), jnp.float32),
                  jax.ShapeDtypeStruct((NW, N), jnp.float32)),
        mesh=plsc.VectorSubcoreMesh(core_axis_name="c", subcore_axis_name="s",
                                    num_cores=NC, num_subcores=NS),
        scratch_types=[
            pltpu.VMEM_SHARED((NP, D), jnp.float32),
            pltpu.VMEM((N,), jnp.float32),
            pltpu.VMEM((N,), jnp.float32),
            pltpu.VMEM((EC,), jnp.int32),
            pltpu.VMEM((EC,), jnp.int32),
            pltpu.VMEM((NBUF, G, D), jnp.float32),
            pltpu.VMEM((N,), jnp.float32),
        ] + [pltpu.SemaphoreType.DMA] * (2 * NBUF),
        compiler_params=pltpu.CompilerParams(needs_layout_passes=False),
    )(_sc_edge_body)


def _sc_edge(*args):
    return _sc_edge_kernel()(*args)


def _tc_embed_body(x_ref, W_ref, asrc_ref, adst_ref, h_ref, s_ref, d_ref):
    h = jnp.dot(x_ref[...], W_ref[...], preferred_element_type=jnp.float32)
    h_ref[...] = h
    s_ref[...] = jnp.sum(h * asrc_ref[...], axis=1, keepdims=True)
    d_ref[...] = jnp.sum(h * adst_ref[...], axis=1, keepdims=True)


def _tc_embed(x, W, asrc, adst):
    return pl.pallas_call(
        _tc_embed_body,
        out_shape=[
            jax.ShapeDtypeStruct((N, D), jnp.float32),
            jax.ShapeDtypeStruct((N, 1), jnp.float32),
            jax.ShapeDtypeStruct((N, 1), jnp.float32),
        ],
    )(x, W, asrc, adst)


def _den_col(den_blk):
    ones = jnp.ones((NW, 1), jnp.float32)
    return jax.lax.dot_general(den_blk, ones, (((0,), (0,)), ((), ())),
                               preferred_element_type=jnp.float32)


def _tc_combine_body(acc_ref, den_ref, b_ref, W_ref, asrc_ref, adst_ref,
                     h_ref, s_ref, d_ref):
    num = acc_ref[0, :N] + acc_ref[1, :N]
    den = _den_col(den_ref[...])
    res = num / (den + 1e-16) + b_ref[...]
    hin = jnp.maximum(res, 0.0)
    h = jnp.dot(hin, W_ref[...], preferred_element_type=jnp.float32)
    h_ref[...] = h
    s_ref[...] = jnp.sum(h * asrc_ref[...], axis=1, keepdims=True)
    d_ref[...] = jnp.sum(h * adst_ref[...], axis=1, keepdims=True)


def _tc_combine(acc, den, b, W, asrc, adst):
    return pl.pallas_call(
        _tc_combine_body,
        out_shape=[
            jax.ShapeDtypeStruct((N, D), jnp.float32),
            jax.ShapeDtypeStruct((N, 1), jnp.float32),
            jax.ShapeDtypeStruct((N, 1), jnp.float32),
        ],
    )(acc, den, b, W, asrc, adst)


def _tc_head_body(acc_ref, den_ref, b_ref, lw1_ref, lb1_ref, lw2_ref,
                  lb2_ref, o_ref):
    num = acc_ref[0, :N] + acc_ref[1, :N]
    den = _den_col(den_ref[...])
    res = num / (den + 1e-16) + b_ref[...]
    t = jnp.dot(res, lw1_ref[...], preferred_element_type=jnp.float32)
    t = jnp.maximum(t + lb1_ref[...], 0.0)
    o = jnp.dot(t, lw2_ref[...], preferred_element_type=jnp.float32)
    o_ref[...] = o + lb2_ref[...]


def _tc_head(acc, den, b, lw1, lb1, lw2, lb2):
    return pl.pallas_call(
        _tc_head_body,
        out_shape=jax.ShapeDtypeStruct((N, 1), jnp.float32),
    )(acc, den, b, lw1, lb1, lw2, lb2)


def kernel(x, edge_index, W1, a_src1, a_dst1, b1, W2, a_src2, a_dst2, b2,
           lw1, lb1, lw2, lb2):
    src = edge_index[0].astype(jnp.int32)
    dst = edge_index[1].astype(jnp.int32)
    zeros = jnp.zeros((RPT, D), jnp.float32)
    zn = jnp.zeros((N,), jnp.float32)

    h1, as1, ad1 = _tc_embed(x, W1, a_src1, a_dst1)
    acc1, den1 = _sc_edge(h1, as1.reshape(N), ad1.reshape(N), src, dst,
                          zeros, zn)
    h2, as2, ad2 = _tc_combine(acc1, den1, b1.reshape(1, D), W2,
                               a_src2, a_dst2)
    acc2, den2 = _sc_edge(h2, as2.reshape(N), ad2.reshape(N), src, dst,
                          zeros, zn)
    return _tc_head(acc2, den2, b2.reshape(1, D),
                    lw1, lb1.reshape(1, HID), lw2, lb2.reshape(1, 1))
